# baseline (speedup 1.0000x reference)
import jax
import jax.numpy as jnp
from jax import lax
from jax.experimental import pallas as pl
from jax.experimental.pallas import tpu as pltpu
from jax.experimental.pallas import tpu_sc as plsc
from jax._src.pallas import mpmd as _mpmd

RES = 256
NUM_CELLS = RES ** 3
N_UPDATE = NUM_CELLS // 4
EMA_DECAY = 0.95
OCC_THRE = 0.01

NW = 32
J_PER_W = N_UPDATE // NW
W = 8192
N_WIN = J_PER_W // W
L = 16

_mesh = plsc.VectorSubcoreMesh(core_axis_name="c", subcore_axis_name="s")


def _wid():
    return lax.axis_index("s") * 2 + lax.axis_index("c")


def _k1_body(occs_hbm, idx_hbm, occ_hbm, uall_hbm,
             idx_v, occ_v, g_v, u_v, sem):
    base0 = _wid() * J_PER_W

    def win(w, carry):
        base = base0 + w * W
        pltpu.sync_copy(idx_hbm.at[pl.ds(base, W)], idx_v)
        pltpu.sync_copy(occ_hbm.at[pl.ds(base, W)], occ_v)
        pltpu.async_copy(occs_hbm.at[idx_v], g_v, sem).wait()

        def inner(i, c):
            s = pl.ds(i * L, L)
            u_v[s] = jnp.maximum(g_v[s] * EMA_DECAY, occ_v[s])
            return c

        lax.fori_loop(0, W // L, inner, 0)
        pltpu.sync_copy(u_v, uall_hbm.at[pl.ds(base, W)])
        return carry

    lax.fori_loop(0, N_WIN, win, 0)


_k1 = pl.kernel(
    _k1_body,
    out_type=(jax.ShapeDtypeStruct((N_UPDATE,), jnp.float32),),
    mesh=_mesh,
    scratch_types=[
        pltpu.VMEM((W,), jnp.int32),
        pltpu.VMEM((W,), jnp.float32),
        pltpu.VMEM((W,), jnp.float32),
        pltpu.VMEM((W,), jnp.float32),
        pltpu.SemaphoreType.DMA,
    ],
    name="occ_k1_gather_update",
)


PAD = 512


def _k4_body(sidx_hbm, su_hbm, occs_in, occs_out,
             ni_v, sv_v, ti_v, sem):
    base0 = _wid() * J_PER_W
    iota = lax.iota(jnp.int32, L)
    dummy = NUM_CELLS + _wid() * L + iota

    def win(w, carry):
        base = base0 + w * W
        pltpu.sync_copy(sidx_hbm.at[pl.ds(base, W)], ni_v.at[pl.ds(0, W)])
        pltpu.sync_copy(su_hbm.at[pl.ds(base, W)], sv_v)

        @pl.when(base + W < N_UPDATE)
        def _():
            pltpu.sync_copy(sidx_hbm.at[pl.ds(base + W, L)],
                            ni_v.at[pl.ds(W, L)])

        @pl.when(base + W >= N_UPDATE)
        def _():
            ni_v[pl.ds(W, L)] = jnp.full((L,), -1, jnp.int32)

        def inner(i, c):
            o = i * L
            a = ni_v[pl.ds(o, L)]
            b = plsc.load_gather(ni_v, [o + 1 + iota])
            ti_v[pl.ds(o, L)] = jnp.where(a != b, a, dummy)
            return c

        lax.fori_loop(0, W // L, inner, 0)
        pltpu.async_copy(sv_v, occs_out.at[ti_v], sem).wait()
        return carry

    lax.fori_loop(0, N_WIN, win, 0)


_k4 = _mpmd._mpmd_map(
    [(_mesh, _k4_body)],
    (jax.ShapeDtypeStruct((NUM_CELLS + PAD,), jnp.float32),),
    input_output_aliases={2: 0},
    scratch_types=[
        pltpu.VMEM((W + L,), jnp.int32),
        pltpu.VMEM((W,), jnp.float32),
        pltpu.VMEM((W,), jnp.int32),
        pltpu.SemaphoreType.DMA,
    ],
    compiler_params=pltpu.CompilerParams(needs_layout_passes=False),
    name="occ_k4_sorted_scatter",
)


BLK = 1 << 20
N_BLK = NUM_CELLS // BLK


def _sum_body(x_ref, o_ref):
    @pl.when(pl.program_id(0) == 0)
    def _():
        o_ref[0, 0] = 0.0

    o_ref[0, 0] += jnp.sum(x_ref[...])


_ksum = pl.pallas_call(
    _sum_body,
    out_shape=jax.ShapeDtypeStruct((1, 1), jnp.float32),
    grid=(N_BLK,),
    in_specs=[pl.BlockSpec((BLK,), lambda i: (i,))],
    out_specs=pl.BlockSpec((1, 1), lambda i: (0, 0),
                           memory_space=pltpu.SMEM),
    name="occ_k5_sum",
)


def _bin_body(t_ref, x_ref, o_ref):
    o_ref[...] = x_ref[...] > t_ref[0, 0]


_kbin = pl.pallas_call(
    _bin_body,
    out_shape=jax.ShapeDtypeStruct((NUM_CELLS,), jnp.bool_),
    grid=(N_BLK,),
    in_specs=[
        pl.BlockSpec(memory_space=pltpu.SMEM),
        pl.BlockSpec((BLK,), lambda i: (i,)),
    ],
    out_specs=pl.BlockSpec((BLK,), lambda i: (i,)),
    name="occ_k6_binary",
)


def kernel(occs, indices, occ):
    (uall,) = _k1(occs, indices, occ)
    sidx, su = lax.sort((indices, uall), dimension=0, num_keys=1,
                        is_stable=False)
    occs_pad = jnp.concatenate([occs, jnp.zeros((PAD,), jnp.float32)])
    (occs_new_pad,) = _k4(sidx, su, occs_pad)
    occs_new = occs_new_pad[:NUM_CELLS]
    total = _ksum(occs_new)
    thresh = jnp.minimum(total[0, 0] / NUM_CELLS, OCC_THRE)
    binary = _kbin(thresh.reshape(1, 1), occs_new)
    return occs_new, binary.reshape(RES, RES, RES)


# speedup vs baseline: 2.2315x; 2.2315x over previous
import jax
import jax.numpy as jnp
from jax import lax
from jax.experimental import pallas as pl
from jax.experimental.pallas import tpu as pltpu
from jax.experimental.pallas import tpu_sc as plsc
from jax._src.pallas import mpmd as _mpmd

RES = 256
NUM_CELLS = RES ** 3
N_UPDATE = NUM_CELLS // 4
EMA_DECAY = 0.95
OCC_THRE = 0.01

NW = 32
J_PER_W = N_UPDATE // NW
W = 8192
N_WIN = J_PER_W // W
L = 16

_mesh = plsc.VectorSubcoreMesh(core_axis_name="c", subcore_axis_name="s")


def _wid():
    return lax.axis_index("s") * 2 + lax.axis_index("c")


def _k1_body(occs_hbm, idx_hbm, occ_hbm, uall_hbm,
             idx_v, occ_v, g_v, u_v, sem):
    base0 = _wid() * J_PER_W

    def win(w, carry):
        base = base0 + w * W
        pltpu.sync_copy(idx_hbm.at[pl.ds(base, W)], idx_v)
        pltpu.sync_copy(occ_hbm.at[pl.ds(base, W)], occ_v)
        pltpu.async_copy(occs_hbm.at[idx_v], g_v, sem).wait()

        def inner(i, c):
            s = pl.ds(i * L, L)
            u_v[s] = jnp.maximum(g_v[s] * EMA_DECAY, occ_v[s])
            return c

        lax.fori_loop(0, W // L, inner, 0)
        pltpu.sync_copy(u_v, uall_hbm.at[pl.ds(base, W)])
        return carry

    lax.fori_loop(0, N_WIN, win, 0)


_k1 = pl.kernel(
    _k1_body,
    out_type=(jax.ShapeDtypeStruct((N_UPDATE,), jnp.float32),),
    mesh=_mesh,
    scratch_types=[
        pltpu.VMEM((W,), jnp.int32),
        pltpu.VMEM((W,), jnp.float32),
        pltpu.VMEM((W,), jnp.float32),
        pltpu.VMEM((W,), jnp.float32),
        pltpu.SemaphoreType.DMA,
    ],
    name="occ_k1_gather_update",
)


PAD = NW * W


def _k4_body(sidx_hbm, su_hbm, occs_in, occs_out,
             ni_v, sv_v, ti_v, sem):
    base0 = _wid() * J_PER_W
    iota = lax.iota(jnp.int32, L)
    dummy0 = NUM_CELLS + _wid() * W

    def win(w, carry):
        base = base0 + w * W
        pltpu.sync_copy(sidx_hbm.at[pl.ds(base, W)], ni_v.at[pl.ds(0, W)])
        pltpu.sync_copy(su_hbm.at[pl.ds(base, W)], sv_v)

        @pl.when(base + W < N_UPDATE)
        def _():
            pltpu.sync_copy(sidx_hbm.at[pl.ds(base + W, L)],
                            ni_v.at[pl.ds(W, L)])

        @pl.when(base + W >= N_UPDATE)
        def _():
            ni_v[pl.ds(W, L)] = jnp.full((L,), -1, jnp.int32)

        def inner(i, c):
            o = i * L
            a = ni_v[pl.ds(o, L)]
            b = plsc.load_gather(ni_v, [o + 1 + iota])
            ti_v[pl.ds(o, L)] = jnp.where(a != b, a, dummy0 + o + iota)
            return c

        lax.fori_loop(0, W // L, inner, 0)
        pltpu.async_copy(sv_v, occs_out.at[ti_v], sem).wait()
        return carry

    lax.fori_loop(0, N_WIN, win, 0)


_k4 = _mpmd._mpmd_map(
    [(_mesh, _k4_body)],
    (jax.ShapeDtypeStruct((NUM_CELLS + PAD,), jnp.float32),),
    input_output_aliases={2: 0},
    scratch_types=[
        pltpu.VMEM((W + L,), jnp.int32),
        pltpu.VMEM((W,), jnp.float32),
        pltpu.VMEM((W,), jnp.int32),
        pltpu.SemaphoreType.DMA,
    ],
    compiler_params=pltpu.CompilerParams(needs_layout_passes=False),
    name="occ_k4_sorted_scatter",
)


BLK = 1 << 20
N_BLK = NUM_CELLS // BLK


def _sum_body(x_ref, o_ref):
    @pl.when(pl.program_id(0) == 0)
    def _():
        o_ref[0, 0] = 0.0

    o_ref[0, 0] += jnp.sum(x_ref[...])


_ksum = pl.pallas_call(
    _sum_body,
    out_shape=jax.ShapeDtypeStruct((1, 1), jnp.float32),
    grid=(N_BLK,),
    in_specs=[pl.BlockSpec((BLK,), lambda i: (i,))],
    out_specs=pl.BlockSpec((1, 1), lambda i: (0, 0),
                           memory_space=pltpu.SMEM),
    name="occ_k5_sum",
)


def _bin_body(t_ref, x_ref, o_ref):
    o_ref[...] = x_ref[...] > t_ref[0, 0]


_kbin = pl.pallas_call(
    _bin_body,
    out_shape=jax.ShapeDtypeStruct((NUM_CELLS,), jnp.bool_),
    grid=(N_BLK,),
    in_specs=[
        pl.BlockSpec(memory_space=pltpu.SMEM),
        pl.BlockSpec((BLK,), lambda i: (i,)),
    ],
    out_specs=pl.BlockSpec((BLK,), lambda i: (i,)),
    name="occ_k6_binary",
)


def kernel(occs, indices, occ):
    (uall,) = _k1(occs, indices, occ)
    sidx, su = lax.sort((indices, uall), dimension=0, num_keys=1,
                        is_stable=False)
    occs_pad = jnp.concatenate([occs, jnp.zeros((PAD,), jnp.float32)])
    (occs_new_pad,) = _k4(sidx, su, occs_pad)
    occs_new = occs_new_pad[:NUM_CELLS]
    total = _ksum(occs_new)
    thresh = jnp.minimum(total[0, 0] / NUM_CELLS, OCC_THRE)
    binary = _kbin(thresh.reshape(1, 1), occs_new)
    return occs_new, binary.reshape(RES, RES, RES)


# speedup vs baseline: 4.9709x; 2.2276x over previous
import jax
import jax.numpy as jnp
from jax import lax
from jax.experimental import pallas as pl
from jax.experimental.pallas import tpu as pltpu
from jax.experimental.pallas import tpu_sc as plsc

RES = 256
NUM_CELLS = RES ** 3
N_UPDATE = NUM_CELLS // 4
EMA_DECAY = 0.95
OCC_THRE = 0.01

NW = 32
J_PER_W = N_UPDATE // NW
W = 8192
N_WIN = J_PER_W // W
L = 16

_mesh = plsc.VectorSubcoreMesh(core_axis_name="c", subcore_axis_name="s")


def _wid():
    return lax.axis_index("s") * 2 + lax.axis_index("c")


def _k1_body(occs_hbm, idx_hbm, occ_hbm, uall_hbm,
             idx_v, occ_v, g_v, u_v, sem):
    base0 = _wid() * J_PER_W

    def win(w, carry):
        base = base0 + w * W
        pltpu.sync_copy(idx_hbm.at[pl.ds(base, W)], idx_v)
        pltpu.sync_copy(occ_hbm.at[pl.ds(base, W)], occ_v)
        pltpu.async_copy(occs_hbm.at[idx_v], g_v, sem).wait()

        def inner(i, c):
            s = pl.ds(i * L, L)
            u_v[s] = jnp.maximum(g_v[s] * EMA_DECAY, occ_v[s])
            return c

        lax.fori_loop(0, W // L, inner, 0)
        pltpu.sync_copy(u_v, uall_hbm.at[pl.ds(base, W)])
        return carry

    lax.fori_loop(0, N_WIN, win, 0)


_k1 = pl.kernel(
    _k1_body,
    out_type=(jax.ShapeDtypeStruct((N_UPDATE,), jnp.float32),),
    mesh=_mesh,
    scratch_types=[
        pltpu.VMEM((W,), jnp.int32),
        pltpu.VMEM((W,), jnp.float32),
        pltpu.VMEM((W,), jnp.float32),
        pltpu.VMEM((W,), jnp.float32),
        pltpu.SemaphoreType.DMA,
    ],
    name="occ_k1_gather_update",
)


WC = 8192
N_WINT = NUM_CELLS // WC
WIN_PER_W = N_WINT // NW
NB = 2080
UBG = 256
SENT = jnp.int32(0x3FFFFFFF)


def _k4_body(sidx_hbm, su_hbm, occs_hbm, bnd_hbm, out_hbm,
             buf_v, ni_v, nu_v, bv_v):
    w = _wid()
    iota = lax.iota(jnp.int32, L)
    pltpu.sync_copy(bnd_hbm.at[pl.ds(w * WIN_PER_W, 80)], bv_v)

    def win(j, carry):
        wb = (w * WIN_PER_W + j) * WC
        pltpu.sync_copy(occs_hbm.at[pl.ds(wb, WC)], buf_v)
        s0 = jnp.max(plsc.load_gather(bv_v, [jnp.full((L,), j, jnp.int32)]))
        s1 = jnp.max(plsc.load_gather(bv_v,
                                      [jnp.full((L,), j + 1, jnp.int32)]))
        g0 = s0 // L
        g1 = (s1 + L - 1) // L

        def blk(t, c):
            blk0 = g0 + t * UBG
            off = blk0 * L
            pltpu.sync_copy(sidx_hbm.at[pl.ds(off, UBG * L + L)], ni_v)
            pltpu.sync_copy(su_hbm.at[pl.ds(off, UBG * L)], nu_v)

            def grp(gl, c2):
                o = gl * L
                a = ni_v[pl.ds(o, L)]
                b = plsc.load_gather(ni_v, [o + 1 + iota])
                m = (a >= wb) & (a < wb + WC) & (a != b)
                tgt = jnp.where(m, a - wb, 0)
                plsc.store_scatter(buf_v, [tgt], nu_v[pl.ds(o, L)], mask=m)
                return c2

            lax.fori_loop(0, jnp.minimum(UBG, g1 - blk0), grp, 0)
            return c

        lax.fori_loop(0, (g1 - g0 + UBG - 1) // UBG, blk, 0)
        pltpu.sync_copy(buf_v, out_hbm.at[pl.ds(wb, WC)])
        return carry

    lax.fori_loop(0, WIN_PER_W, win, 0)


_k4 = pl.kernel(
    _k4_body,
    out_type=(jax.ShapeDtypeStruct((NUM_CELLS,), jnp.float32),),
    mesh=_mesh,
    scratch_types=[
        pltpu.VMEM((WC,), jnp.float32),
        pltpu.VMEM((UBG * L + L,), jnp.int32),
        pltpu.VMEM((UBG * L,), jnp.float32),
        pltpu.VMEM((80,), jnp.int32),
    ],
    compiler_params=pltpu.CompilerParams(needs_layout_passes=False),
    name="occ_k4_merge_pass",
)


BLK = 1 << 20
N_BLK = NUM_CELLS // BLK


def _sum_body(x_ref, o_ref):
    @pl.when(pl.program_id(0) == 0)
    def _():
        o_ref[0, 0] = 0.0

    o_ref[0, 0] += jnp.sum(x_ref[...])


_ksum = pl.pallas_call(
    _sum_body,
    out_shape=jax.ShapeDtypeStruct((1, 1), jnp.float32),
    grid=(N_BLK,),
    in_specs=[pl.BlockSpec((BLK,), lambda i: (i,))],
    out_specs=pl.BlockSpec((1, 1), lambda i: (0, 0),
                           memory_space=pltpu.SMEM),
    name="occ_k5_sum",
)


def _bin_body(t_ref, x_ref, o_ref):
    o_ref[...] = x_ref[...] > t_ref[0, 0]


_kbin = pl.pallas_call(
    _bin_body,
    out_shape=jax.ShapeDtypeStruct((NUM_CELLS,), jnp.bool_),
    grid=(N_BLK,),
    in_specs=[
        pl.BlockSpec(memory_space=pltpu.SMEM),
        pl.BlockSpec((BLK,), lambda i: (i,)),
    ],
    out_specs=pl.BlockSpec((BLK,), lambda i: (i,)),
    name="occ_k6_binary",
)


def kernel(occs, indices, occ):
    (uall,) = _k1(occs, indices, occ)
    sidx, su = lax.sort((indices, uall), dimension=0, num_keys=1,
                        is_stable=False)
    bnd = jnp.searchsorted(
        sidx, jnp.arange(N_WINT + 1, dtype=jnp.int32) * WC).astype(jnp.int32)
    bnd = jnp.concatenate(
        [bnd, jnp.full((NB - N_WINT - 1,), N_UPDATE, jnp.int32)])
    sidx_p = jnp.concatenate([sidx, jnp.full((UBG * L + L,), SENT)])
    su_p = jnp.concatenate([su, jnp.zeros((UBG * L,), jnp.float32)])
    (occs_new,) = _k4(sidx_p, su_p, occs, bnd)
    total = _ksum(occs_new)
    thresh = jnp.minimum(total[0, 0] / NUM_CELLS, OCC_THRE)
    binary = _kbin(thresh.reshape(1, 1), occs_new)
    return occs_new, binary.reshape(RES, RES, RES)
